# baseline (speedup 1.0000x reference)
import functools

import jax
import jax.numpy as jnp
from jax import lax
from jax.experimental import pallas as pl
from jax.experimental.pallas import tpu as pltpu
from jax.experimental.pallas import tpu_sc as plsc


def _sc_gather(emb, idx_flat):
    n_idx = idx_flat.shape[0]
    e_dim = emb.shape[1]
    info = plsc.get_sparse_core_info()
    n_workers = info.num_cores * info.num_subcores
    per_w = n_idx // n_workers
    chunk = 80
    n_chunks = per_w // chunk

    mesh = plsc.VectorSubcoreMesh(core_axis_name="c", subcore_axis_name="s")

    @functools.partial(
        pl.kernel,
        mesh=mesh,
        out_type=jax.ShapeDtypeStruct((n_idx, e_dim), jnp.float32),
        scratch_types=[
            pltpu.VMEM((per_w,), jnp.int32),
            pltpu.VMEM((chunk, e_dim), jnp.float32),
            pltpu.VMEM((chunk, e_dim), jnp.float32),
            pltpu.SemaphoreType.DMA,
            pltpu.SemaphoreType.DMA,
        ],
    )
    def gather_kernel(emb_hbm, idx_hbm, out_hbm, idx_v, buf0, buf1, sem0, sem1):
        wid = lax.axis_index("s") * info.num_cores + lax.axis_index("c")
        base = wid * per_w
        pltpu.sync_copy(idx_hbm.at[pl.ds(base, per_w)], idx_v)
        bufs = (buf0, buf1)
        sems = (sem0, sem1)

        def start(ci):
            return pltpu.async_copy(
                emb_hbm.at[idx_v.at[pl.ds(ci * chunk, chunk)]],
                bufs[ci % 2],
                sems[ci % 2],
            )

        cps = [None] * n_chunks
        cps[0] = start(0)
        for ci in range(n_chunks):
            if ci + 1 < n_chunks:
                cps[ci + 1] = start(ci + 1)
            cps[ci].wait()
            pltpu.sync_copy(
                bufs[ci % 2], out_hbm.at[pl.ds(base + ci * chunk, chunk)]
            )

    return gather_kernel(emb, idx_flat)


def _lstm_head(xe, xe0, w_x, w_h, b_gates, w_lin_t, b_lin, w_out_t, b_out,
               s_len, b_dim, h_dim):
    def body(xe_ref, xe0_ref, wx_ref, wh_ref, bg_ref, wlin_ref, blin_ref,
             wout_ref, bout_ref, out_ref, h_ref, c_ref, xp_ref):
        t = pl.program_id(0)

        @pl.when(t == 0)
        def _init():
            h_ref[...] = jnp.zeros_like(h_ref)
            c_ref[...] = jnp.zeros_like(c_ref)
            xp_ref[...] = jnp.dot(
                xe0_ref[0].astype(jnp.bfloat16), wx_ref[...],
                preferred_element_type=jnp.float32) + bg_ref[0:1, :]

        pre = (
            xp_ref[...]
            + jnp.dot(h_ref[...], wh_ref[...],
                      preferred_element_type=jnp.float32)
        )
        sg = jnp.tanh(pre[:, : 3 * h_dim])
        i_t = sg[:, 0 * h_dim:1 * h_dim]
        f_t = sg[:, 1 * h_dim:2 * h_dim]
        o_t = sg[:, 2 * h_dim:3 * h_dim]
        g_t = jnp.tanh(pre[:, 3 * h_dim:])
        c_old = c_ref[...]
        c_new = 0.5 * (f_t * c_old + c_old + i_t * g_t + g_t)
        h_new = (0.5 * (o_t + 1.0)) * jnp.tanh(c_new)
        c_ref[...] = c_new
        h_ref[...] = h_new.astype(jnp.bfloat16)

        xp_ref[...] = jnp.dot(
            xe_ref[0].astype(jnp.bfloat16), wx_ref[...],
            preferred_element_type=jnp.float32) + bg_ref[0:1, :]

        @pl.when(t == s_len - 1)
        def _head():
            feat = (
                jnp.dot(h_new, wlin_ref[...], preferred_element_type=jnp.float32)
                + blin_ref[0:1, :]
            )
            out_ref[...] = (
                jnp.dot(feat, wout_ref[...], preferred_element_type=jnp.float32)
                + bout_ref[0:1, :]
            )

    e_dim = xe.shape[2]
    nxt = lambda t: (jnp.minimum(t + 1, s_len - 1), 0, 0)
    return pl.pallas_call(
        body,
        grid=(s_len,),
        in_specs=[
            pl.BlockSpec((1, b_dim, e_dim), nxt),
            pl.BlockSpec((1, b_dim, e_dim), lambda t: (0, 0, 0)),
            pl.BlockSpec(w_x.shape, lambda t: (0, 0)),
            pl.BlockSpec(w_h.shape, lambda t: (0, 0)),
            pl.BlockSpec(b_gates.shape, lambda t: (0, 0)),
            pl.BlockSpec(w_lin_t.shape, lambda t: (0, 0)),
            pl.BlockSpec(b_lin.shape, lambda t: (0, 0)),
            pl.BlockSpec(w_out_t.shape, lambda t: (0, 0)),
            pl.BlockSpec(b_out.shape, lambda t: (0, 0)),
        ],
        out_specs=pl.BlockSpec((b_dim, 128), lambda t: (0, 0)),
        out_shape=jax.ShapeDtypeStruct((b_dim, 128), jnp.float32),
        scratch_shapes=[
            pltpu.VMEM((b_dim, h_dim), jnp.bfloat16),
            pltpu.VMEM((b_dim, h_dim), jnp.float32),
            pltpu.VMEM((b_dim, 4 * h_dim), jnp.float32),
        ],
    )(xe, xe0, w_x, w_h, b_gates, w_lin_t, b_lin, w_out_t, b_out)


def _pack_gates(w, h_dim, scale_axis):
    i, f, g, o = jnp.split(w, 4, axis=scale_axis)
    return jnp.concatenate([0.5 * i, 0.5 * f, 0.5 * o, g], axis=scale_axis)


def kernel(x, emb, W_ih, W_hh, b_ih, b_hh, W_lin, b_lin, W_out, b_out):
    s_len, b_dim = x.shape
    h_dim = W_hh.shape[1]
    e_dim = emb.shape[1]
    c_dim = W_out.shape[0]

    idx_flat = x.reshape(-1).astype(jnp.int32)
    xe_flat = _sc_gather(emb, idx_flat)
    xe = xe_flat.reshape(s_len, b_dim, e_dim)

    w_x = _pack_gates(W_ih.T, h_dim, 1).astype(jnp.bfloat16)
    w_h = _pack_gates(W_hh.T, h_dim, 1).astype(jnp.bfloat16)
    b_g = _pack_gates(b_ih + b_hh, h_dim, 0)
    b_gates = jnp.broadcast_to(b_g[None, :], (8, 4 * h_dim))
    w_lin_t = W_lin.T
    b_lin_b = jnp.broadcast_to(b_lin[None, :], (8, h_dim))
    w_out_t = jnp.zeros((h_dim, 128), jnp.float32).at[:, :c_dim].set(W_out.T)
    b_out_b = jnp.broadcast_to(
        jnp.zeros((128,), jnp.float32).at[:c_dim].set(b_out)[None, :], (8, 128)
    )

    out_pad = _lstm_head(xe, xe[:1], w_x, w_h, b_gates, w_lin_t, b_lin_b,
                         w_out_t, b_out_b, s_len, b_dim, h_dim)
    return out_pad[:, :c_dim]


# speedup vs baseline: 1.0002x; 1.0002x over previous
import functools

import jax
import jax.numpy as jnp
from jax import lax
from jax.experimental import pallas as pl
from jax.experimental.pallas import tpu as pltpu
from jax.experimental.pallas import tpu_sc as plsc


def _sc_gather(emb, idx_flat):
    n_idx = idx_flat.shape[0]
    e_dim = emb.shape[1]
    info = plsc.get_sparse_core_info()
    n_workers = info.num_cores * info.num_subcores
    per_w = n_idx // n_workers
    chunk = 80
    n_chunks = per_w // chunk

    mesh = plsc.VectorSubcoreMesh(core_axis_name="c", subcore_axis_name="s")

    @functools.partial(
        pl.kernel,
        mesh=mesh,
        out_type=jax.ShapeDtypeStruct((n_idx, e_dim), jnp.float32),
        scratch_types=[
            pltpu.VMEM((per_w,), jnp.int32),
            pltpu.VMEM((chunk, e_dim), jnp.float32),
            pltpu.VMEM((chunk, e_dim), jnp.float32),
            pltpu.SemaphoreType.DMA,
            pltpu.SemaphoreType.DMA,
        ],
    )
    def gather_kernel(emb_hbm, idx_hbm, out_hbm, idx_v, buf0, buf1, sem0, sem1):
        wid = lax.axis_index("s") * info.num_cores + lax.axis_index("c")
        base = wid * per_w
        pltpu.sync_copy(idx_hbm.at[pl.ds(base, per_w)], idx_v)
        bufs = (buf0, buf1)
        sems = (sem0, sem1)

        def start(ci):
            return pltpu.async_copy(
                emb_hbm.at[idx_v.at[pl.ds(ci * chunk, chunk)]],
                bufs[ci % 2],
                sems[ci % 2],
            )

        cps = [None] * n_chunks
        cps[0] = start(0)
        for ci in range(n_chunks):
            if ci + 1 < n_chunks:
                cps[ci + 1] = start(ci + 1)
            cps[ci].wait()
            pltpu.sync_copy(
                bufs[ci % 2], out_hbm.at[pl.ds(base + ci * chunk, chunk)]
            )

    return gather_kernel(emb, idx_flat)


def _lstm_head(xe, xe0, w_x, w_h, b_gates, w_lin_t, b_lin, w_out_t, b_out,
               s_len, b_dim, h_dim):
    def body(xe_ref, xe0_ref, wx_ref, wh_ref, bg_ref, wlin_ref, blin_ref,
             wout_ref, bout_ref, out_ref, h_ref, c_ref, xp_ref):
        t = pl.program_id(0)

        @pl.when(t == 0)
        def _init():
            h_ref[...] = jnp.zeros_like(h_ref)
            c_ref[...] = jnp.zeros_like(c_ref)
            xp_ref[...] = jnp.dot(
                xe0_ref[0].astype(jnp.bfloat16), wx_ref[...],
                preferred_element_type=jnp.float32) + bg_ref[0:1, :]

        pre = (
            xp_ref[...]
            + jnp.dot(h_ref[...].astype(jnp.bfloat16), wh_ref[...],
                      preferred_element_type=jnp.float32)
        )
        sg = jnp.tanh(pre[:, : 3 * h_dim])
        i_t = sg[:, 0 * h_dim:1 * h_dim]
        f_t = sg[:, 1 * h_dim:2 * h_dim]
        o_t = sg[:, 2 * h_dim:3 * h_dim]
        g_t = jnp.tanh(pre[:, 3 * h_dim:])
        c_old = c_ref[...]
        c_new = 0.5 * (f_t * c_old + c_old + i_t * g_t + g_t)
        h_new = (0.5 * (o_t + 1.0)) * jnp.tanh(c_new)
        c_ref[...] = c_new
        h_ref[...] = h_new

        xp_ref[...] = jnp.dot(
            xe_ref[0].astype(jnp.bfloat16), wx_ref[...],
            preferred_element_type=jnp.float32) + bg_ref[0:1, :]

        @pl.when(t == s_len - 1)
        def _head():
            feat = (
                jnp.dot(h_new, wlin_ref[...], preferred_element_type=jnp.float32)
                + blin_ref[0:1, :]
            )
            out_ref[...] = (
                jnp.dot(feat, wout_ref[...], preferred_element_type=jnp.float32)
                + bout_ref[0:1, :]
            )

    e_dim = xe.shape[2]
    nxt = lambda t: (jnp.minimum(t + 1, s_len - 1), 0, 0)
    return pl.pallas_call(
        body,
        grid=(s_len,),
        in_specs=[
            pl.BlockSpec((1, b_dim, e_dim), nxt),
            pl.BlockSpec((1, b_dim, e_dim), lambda t: (0, 0, 0)),
            pl.BlockSpec(w_x.shape, lambda t: (0, 0)),
            pl.BlockSpec(w_h.shape, lambda t: (0, 0)),
            pl.BlockSpec(b_gates.shape, lambda t: (0, 0)),
            pl.BlockSpec(w_lin_t.shape, lambda t: (0, 0)),
            pl.BlockSpec(b_lin.shape, lambda t: (0, 0)),
            pl.BlockSpec(w_out_t.shape, lambda t: (0, 0)),
            pl.BlockSpec(b_out.shape, lambda t: (0, 0)),
        ],
        out_specs=pl.BlockSpec((b_dim, 128), lambda t: (0, 0)),
        out_shape=jax.ShapeDtypeStruct((b_dim, 128), jnp.float32),
        scratch_shapes=[
            pltpu.VMEM((b_dim, h_dim), jnp.float32),
            pltpu.VMEM((b_dim, h_dim), jnp.float32),
            pltpu.VMEM((b_dim, 4 * h_dim), jnp.float32),
        ],
    )(xe, xe0, w_x, w_h, b_gates, w_lin_t, b_lin, w_out_t, b_out)


def _pack_gates(w, h_dim, scale_axis):
    i, f, g, o = jnp.split(w, 4, axis=scale_axis)
    return jnp.concatenate([0.5 * i, 0.5 * f, 0.5 * o, g], axis=scale_axis)


def kernel(x, emb, W_ih, W_hh, b_ih, b_hh, W_lin, b_lin, W_out, b_out):
    s_len, b_dim = x.shape
    h_dim = W_hh.shape[1]
    e_dim = emb.shape[1]
    c_dim = W_out.shape[0]

    idx_flat = x.reshape(-1).astype(jnp.int32)
    xe_flat = _sc_gather(emb, idx_flat)
    xe = xe_flat.reshape(s_len, b_dim, e_dim)

    w_x = _pack_gates(W_ih.T, h_dim, 1).astype(jnp.bfloat16)
    w_h = _pack_gates(W_hh.T, h_dim, 1).astype(jnp.bfloat16)
    b_g = _pack_gates(b_ih + b_hh, h_dim, 0)
    b_gates = jnp.broadcast_to(b_g[None, :], (8, 4 * h_dim))
    w_lin_t = W_lin.T
    b_lin_b = jnp.broadcast_to(b_lin[None, :], (8, h_dim))
    w_out_t = jnp.zeros((h_dim, 128), jnp.float32).at[:, :c_dim].set(W_out.T)
    b_out_b = jnp.broadcast_to(
        jnp.zeros((128,), jnp.float32).at[:c_dim].set(b_out)[None, :], (8, 128)
    )

    out_pad = _lstm_head(xe, xe[:1], w_x, w_h, b_gates, w_lin_t, b_lin_b,
                         w_out_t, b_out_b, s_len, b_dim, h_dim)
    return out_pad[:, :c_dim]


# speedup vs baseline: 1.0891x; 1.0889x over previous
import functools

import jax
import jax.numpy as jnp
from jax import lax
from jax.experimental import pallas as pl
from jax.experimental.pallas import tpu as pltpu
from jax.experimental.pallas import tpu_sc as plsc


def _sc_gather(emb, idx_flat):
    n_idx = idx_flat.shape[0]
    e_dim = emb.shape[1]
    info = plsc.get_sparse_core_info()
    n_workers = info.num_cores * info.num_subcores
    per_w = n_idx // n_workers
    chunk = 80
    n_chunks = per_w // chunk

    mesh = plsc.VectorSubcoreMesh(core_axis_name="c", subcore_axis_name="s")

    @functools.partial(
        pl.kernel,
        mesh=mesh,
        out_type=jax.ShapeDtypeStruct((n_idx, e_dim), jnp.float32),
        scratch_types=[
            pltpu.VMEM((per_w,), jnp.int32),
            pltpu.VMEM((chunk, e_dim), jnp.float32),
            pltpu.VMEM((chunk, e_dim), jnp.float32),
            pltpu.SemaphoreType.DMA,
            pltpu.SemaphoreType.DMA,
        ],
    )
    def gather_kernel(emb_hbm, idx_hbm, out_hbm, idx_v, buf0, buf1, sem0, sem1):
        wid = lax.axis_index("s") * info.num_cores + lax.axis_index("c")
        base = wid * per_w
        pltpu.sync_copy(idx_hbm.at[pl.ds(base, per_w)], idx_v)
        bufs = (buf0, buf1)
        sems = (sem0, sem1)

        def start(ci):
            return pltpu.async_copy(
                emb_hbm.at[idx_v.at[pl.ds(ci * chunk, chunk)]],
                bufs[ci % 2],
                sems[ci % 2],
            )

        cps = [None] * n_chunks
        cps[0] = start(0)
        for ci in range(n_chunks):
            if ci + 1 < n_chunks:
                cps[ci + 1] = start(ci + 1)
            cps[ci].wait()
            pltpu.sync_copy(
                bufs[ci % 2], out_hbm.at[pl.ds(base + ci * chunk, chunk)]
            )

    return gather_kernel(emb, idx_flat)


def _lstm_head(xe, xe0, w_x, w_h, b_gates, w_lin_t, b_lin, w_out_t, b_out,
               s_len, b_dim, h_dim):
    def body(xe_ref, xe0_ref, wx_ref, wh_ref, bg_ref, wlin_ref, blin_ref,
             wout_ref, bout_ref, out_ref, h_ref, c_ref, xp_ref):
        t = pl.program_id(0)

        @pl.when(t == 0)
        def _init():
            h_ref[...] = jnp.zeros_like(h_ref)
            c_ref[...] = jnp.zeros_like(c_ref)
            xp_ref[...] = jnp.dot(
                xe0_ref[0].astype(jnp.bfloat16), wx_ref[...],
                preferred_element_type=jnp.float32)

        pre = (
            xp_ref[...]
            + jnp.dot(h_ref[...].astype(jnp.bfloat16), wh_ref[...],
                      preferred_element_type=jnp.float32)
            + bg_ref[0:1, :]
        )
        sg = jnp.tanh(pre[:, : 3 * h_dim])
        i_t = sg[:, 0 * h_dim:1 * h_dim]
        f_t = sg[:, 1 * h_dim:2 * h_dim]
        o_t = sg[:, 2 * h_dim:3 * h_dim]
        g_t = jnp.tanh(pre[:, 3 * h_dim:])
        c_old = c_ref[...]
        c_new = 0.5 * (f_t * c_old + c_old + i_t * g_t + g_t)
        h_new = (0.5 * (o_t + 1.0)) * jnp.tanh(c_new)
        c_ref[...] = c_new
        h_ref[...] = h_new

        xp_ref[...] = jnp.dot(
            xe_ref[0].astype(jnp.bfloat16), wx_ref[...],
            preferred_element_type=jnp.float32)

        @pl.when(t == s_len - 1)
        def _head():
            feat = (
                jnp.dot(h_new, wlin_ref[...], preferred_element_type=jnp.float32)
                + blin_ref[0:1, :]
            )
            out_ref[...] = (
                jnp.dot(feat, wout_ref[...], preferred_element_type=jnp.float32)
                + bout_ref[0:1, :]
            )

    e_dim = xe.shape[2]
    nxt = lambda t: (jnp.minimum(t + 1, s_len - 1), 0, 0)
    return pl.pallas_call(
        body,
        grid=(s_len,),
        in_specs=[
            pl.BlockSpec((1, b_dim, e_dim), nxt),
            pl.BlockSpec((1, b_dim, e_dim), lambda t: (0, 0, 0)),
            pl.BlockSpec(w_x.shape, lambda t: (0, 0)),
            pl.BlockSpec(w_h.shape, lambda t: (0, 0)),
            pl.BlockSpec(b_gates.shape, lambda t: (0, 0)),
            pl.BlockSpec(w_lin_t.shape, lambda t: (0, 0)),
            pl.BlockSpec(b_lin.shape, lambda t: (0, 0)),
            pl.BlockSpec(w_out_t.shape, lambda t: (0, 0)),
            pl.BlockSpec(b_out.shape, lambda t: (0, 0)),
        ],
        out_specs=pl.BlockSpec((b_dim, 128), lambda t: (0, 0)),
        out_shape=jax.ShapeDtypeStruct((b_dim, 128), jnp.float32),
        scratch_shapes=[
            pltpu.VMEM((b_dim, h_dim), jnp.float32),
            pltpu.VMEM((b_dim, h_dim), jnp.float32),
            pltpu.VMEM((b_dim, 4 * h_dim), jnp.float32),
        ],
    )(xe, xe0, w_x, w_h, b_gates, w_lin_t, b_lin, w_out_t, b_out)


def _pack_gates(w, h_dim, scale_axis):
    i, f, g, o = jnp.split(w, 4, axis=scale_axis)
    return jnp.concatenate([0.5 * i, 0.5 * f, 0.5 * o, g], axis=scale_axis)


def kernel(x, emb, W_ih, W_hh, b_ih, b_hh, W_lin, b_lin, W_out, b_out):
    s_len, b_dim = x.shape
    h_dim = W_hh.shape[1]
    e_dim = emb.shape[1]
    c_dim = W_out.shape[0]

    idx_flat = x.reshape(-1).astype(jnp.int32)
    xe_flat = _sc_gather(emb, idx_flat)
    xe = xe_flat.reshape(s_len, b_dim, e_dim)

    w_x = _pack_gates(W_ih.T, h_dim, 1).astype(jnp.bfloat16)
    w_h = _pack_gates(W_hh.T, h_dim, 1).astype(jnp.bfloat16)
    b_g = _pack_gates(b_ih + b_hh, h_dim, 0)
    b_gates = jnp.broadcast_to(b_g[None, :], (8, 4 * h_dim))
    w_lin_t = W_lin.T
    b_lin_b = jnp.broadcast_to(b_lin[None, :], (8, h_dim))
    w_out_t = jnp.zeros((h_dim, 128), jnp.float32).at[:, :c_dim].set(W_out.T)
    b_out_b = jnp.broadcast_to(
        jnp.zeros((128,), jnp.float32).at[:c_dim].set(b_out)[None, :], (8, 128)
    )

    out_pad = _lstm_head(xe, xe[:1], w_x, w_h, b_gates, w_lin_t, b_lin_b,
                         w_out_t, b_out_b, s_len, b_dim, h_dim)
    return out_pad[:, :c_dim]
